# baseline (speedup 1.0000x reference)
import functools

import jax
import jax.numpy as jnp
from jax import lax
from jax.experimental import pallas as pl
from jax.experimental.pallas import tpu as pltpu
from jax.experimental.pallas import tpu_sc as plsc

KW_VOCAB = 100
N_CANDIDATES = 100
N_TRIPLES = 1000000
BATCH = 4096

_NC, _NS, _L = 2, 16, 16
_NW = _NC * _NS
_BPW = BATCH // _NW
_CP = 104
_TP = 8
_FIRE = 8


def _sc_retrieve(kw0_hbm, kw1_hbm, kwsm_hbm, triple_hbm, out_hbm,
                 kw0_v, kw1_v, flat_v, cand_v, trip_v, sem_a, sem_b):
    wid = lax.axis_index("s") * _NC + lax.axis_index("c")
    base = wid * _BPW

    pltpu.sync_copy(kw0_hbm.at[pl.ds(base, _BPW)], kw0_v)
    pltpu.sync_copy(kw1_hbm.at[pl.ds(base, _BPW)], kw1_v)

    for i in range(_BPW // _L):
        sl = pl.ds(i * _L, _L)
        flat_v[sl] = kw0_v[sl] * KW_VOCAB + kw1_v[sl]

    pltpu.async_copy(kwsm_hbm.at[flat_v], cand_v, sem_a).wait()

    def body(t, carry):
        s0 = t * _FIRE
        copies = [
            pltpu.async_copy(triple_hbm.at[cand_v.at[s0 + j]],
                             trip_v.at[s0 + j], sem_b)
            for j in range(_FIRE)
        ]
        for c in copies:
            c.wait()
        return carry

    lax.fori_loop(0, _BPW // _FIRE, body, 0)

    pltpu.sync_copy(trip_v, out_hbm.at[pl.ds(base, _BPW)])


def kernel(key_word, triple, kw_to_sm):
    kw32 = key_word.astype(jnp.int32)
    kw0 = kw32[:, 0]
    kw1 = kw32[:, 1]
    spread = (jnp.arange(10000, dtype=jnp.int32)[:, None] * 4
              + jnp.arange(4, dtype=jnp.int32)[None, :]) % N_TRIPLES
    kwsm_p = jnp.concatenate([kw_to_sm.astype(jnp.int32), spread], axis=1)
    trip_p = jnp.pad(triple.astype(jnp.int32), ((0, 0), (0, _TP - 3)))

    mesh = plsc.VectorSubcoreMesh(core_axis_name="c", subcore_axis_name="s")
    run = functools.partial(
        pl.kernel,
        mesh=mesh,
        compiler_params=pltpu.CompilerParams(use_tc_tiling_on_sc=False),
        out_type=jax.ShapeDtypeStruct((BATCH, _CP, _TP), jnp.int32),
        scratch_types=[
            pltpu.VMEM((_BPW,), jnp.int32),
            pltpu.VMEM((_BPW,), jnp.int32),
            pltpu.VMEM((_BPW,), jnp.int32),
            pltpu.VMEM((_BPW, _CP), jnp.int32),
            pltpu.VMEM((_BPW, _CP, _TP), jnp.int32),
            pltpu.SemaphoreType.DMA,
            pltpu.SemaphoreType.DMA,
        ],
    )(_sc_retrieve)
    out_p = run(kw0, kw1, kwsm_p, trip_p)
    return out_p[:, :N_CANDIDATES, :3]
